# baseline (speedup 1.0000x reference)
import functools

import jax
import jax.numpy as jnp
from jax import lax
from jax.experimental import pallas as pl
from jax.experimental.pallas import tpu as pltpu
from jax.experimental.pallas import tpu_sc as plsc

_NC, _NS = 2, 16
_NW = _NC * _NS
_CH = 256


@functools.lru_cache(maxsize=None)
def _make_gather(l_total, b_total, d):
    per_l = b_total // _CH
    n_units = l_total * per_l
    upw = n_units // _NW
    assert upw * _NW == n_units and per_l * _CH == b_total

    mesh = plsc.VectorSubcoreMesh(core_axis_name="c", subcore_axis_name="s")

    @functools.partial(
        pl.kernel,
        mesh=mesh,
        compiler_params=pltpu.CompilerParams(use_tc_tiling_on_sc=False),
        out_type=jax.ShapeDtypeStruct((l_total, b_total, d), jnp.float32),
        scratch_types=[
            pltpu.VMEM((2, _CH), jnp.int32),
            pltpu.VMEM((2, _CH, d), jnp.float32),
            pltpu.SemaphoreType.DMA,
            pltpu.SemaphoreType.DMA,
            pltpu.SemaphoreType.DMA,
            pltpu.SemaphoreType.DMA,
        ],
    )
    def emb(table_hbm, idx_hbm, out_hbm, idx_v, rows_v, g0, g1, w0, w1):
        wid = lax.axis_index("s") * _NC + lax.axis_index("c")
        u0 = wid * upw
        gsem = (g0, g1)
        wsem = (w0, w1)
        gath = [None, None]
        wrb = [None, None]
        for i in range(upw):
            cur = i % 2
            u = u0 + i
            l = u // per_l
            q = u % per_l
            if wrb[cur] is not None:
                wrb[cur].wait()
            pltpu.sync_copy(idx_hbm.at[l, pl.ds(q * _CH, _CH)],
                            idx_v.at[cur])
            gath[cur] = pltpu.async_copy(
                table_hbm.at[idx_v.at[cur]], rows_v.at[cur], gsem[cur])
            if i > 0:
                prv = 1 - cur
                up = u - 1
                gath[prv].wait()
                wrb[prv] = pltpu.async_copy(
                    rows_v.at[prv],
                    out_hbm.at[up // per_l, pl.ds((up % per_l) * _CH, _CH)],
                    wsem[prv])
        last = (upw - 1) % 2
        ul = u0 + upw - 1
        gath[last].wait()
        pltpu.sync_copy(rows_v.at[last],
                        out_hbm.at[ul // per_l, pl.ds((ul % per_l) * _CH, _CH)])
        if wrb[1 - last] is not None:
            wrb[1 - last].wait()

    return emb


def kernel(token_ids, weight):
    b, l = token_ids.shape
    idx_t = token_ids.T.astype(jnp.int32)
    out = _make_gather(l, b, weight.shape[1])(weight, idx_t)
    return jnp.transpose(out, (1, 0, 2))
